# Initial kernel scaffold; baseline (speedup 1.0000x reference)
import jax
import jax.numpy as jnp
from jax.experimental import pallas as pl


def kernel(hidden_states, target_ids, embedding_weight, logit_scale):
    raise NotImplementedError("write your pallas kernel here")


# speedup vs baseline: 77.7592x; 77.7592x over previous
import functools

import jax
import jax.numpy as jnp
from jax import lax
from jax.experimental import pallas as pl
from jax.experimental.pallas import tpu as pltpu

V = 100000
D = 1024
LOW = 32
NCAND = 2048
CH = 128
AUX = 0.2


def _scan_body(mean_ref, el_ref, out_ref):
    e = el_ref[...]
    nrm = jnp.sqrt(jnp.sum(e * e, axis=1, keepdims=True))
    lg = lax.dot_general(
        e, mean_ref[...],
        (((1,), (1,)), ((), ())),
        preferred_element_type=jnp.float32,
        precision=lax.Precision.HIGHEST,
    )
    out_ref[...] = lg / jnp.maximum(nrm, 1e-12)


def _scan_logits(means, e_low):
    nblk = 10
    bv = V // nblk
    return pl.pallas_call(
        _scan_body,
        grid=(nblk,),
        in_specs=[
            pl.BlockSpec((8, LOW), lambda i: (0, 0)),
            pl.BlockSpec((bv, LOW), lambda i: (i, 0)),
        ],
        out_specs=pl.BlockSpec((bv, 8), lambda i: (i, 0)),
        out_shape=jax.ShapeDtypeStruct((V, 8), jnp.float32),
    )(means, e_low)


def _normalize(x, axis=-1):
    n = jnp.sqrt(jnp.sum(x * x, axis=axis, keepdims=True))
    return x / jnp.maximum(n, 1e-12)


def kernel(hidden_states, target_ids, embedding_weight, logit_scale):
    hs = hidden_states.reshape(-1, hidden_states.shape[-1])
    tid = target_ids.reshape(-1)
    total = hs.shape[0]
    nchunk = total // CH

    h_full = _normalize(hs)
    h_low = _normalize(hs[:, :LOW])
    means = jnp.mean(h_low.reshape(nchunk, CH, LOW), axis=1)

    e_low = embedding_weight[:, :LOW]
    logits = _scan_logits(means, e_low)
    _, top_idx = lax.top_k(logits.T, NCAND)

    scale = 100.0 * jax.nn.sigmoid(logit_scale) + 1.0

    cand = embedding_weight[top_idx]
    candn = _normalize(cand)
    cand_lown = _normalize(cand[:, :, :LOW])

    hf = h_full.reshape(nchunk, CH, D)
    hl = h_low.reshape(nchunk, CH, LOW)
    tidc = tid.reshape(nchunk, CH)

    full_sims = jnp.einsum("ctd,cjd->ctj", hf, candn,
                           precision=lax.Precision.HIGHEST)
    low_sims = jnp.einsum("ctd,cjd->ctj", hl, cand_lown,
                          precision=lax.Precision.HIGHEST)

    tgt = embedding_weight[tid]
    t_full = jnp.sum(h_full * _normalize(tgt), axis=-1).reshape(nchunk, CH)
    t_low = jnp.sum(h_low * _normalize(tgt[:, :LOW]), axis=-1).reshape(nchunk, CH)

    is_t = top_idx[:, None, :] == tidc[:, :, None]
    neg = jnp.float32(-jnp.inf)
    full_sims = jnp.where(is_t, neg, full_sims)
    low_sims = jnp.where(is_t, neg, low_sims)

    def _loss(sims, tsim):
        lg = jnp.concatenate([tsim[:, :, None], sims], axis=2) * scale
        return jnp.sum(jax.nn.logsumexp(lg, axis=2) - lg[:, :, 0])

    main = _loss(full_sims, t_full)
    aux = _loss(low_sims, t_low)
    return (main + AUX * aux) / total


# speedup vs baseline: 78.0177x; 1.0033x over previous
import functools

import jax
import jax.numpy as jnp
from jax import lax
from jax.experimental import pallas as pl
from jax.experimental.pallas import tpu as pltpu
from jax.experimental.pallas import tpu_sc as plsc

V = 100000
D = 1024
LOW = 32
NCAND = 2048
CH = 128
AUX = 0.2


def _scan_body(mean_ref, el_ref, out_ref):
    e = el_ref[...]
    nrm = jnp.sqrt(jnp.sum(e * e, axis=1, keepdims=True))
    lg = lax.dot_general(
        e, mean_ref[...],
        (((1,), (1,)), ((), ())),
        preferred_element_type=jnp.float32,
        precision=lax.Precision.HIGHEST,
    )
    out_ref[...] = lg / jnp.maximum(nrm, 1e-12)


def _scan_logits(means, e_low):
    nblk = 10
    bv = V // nblk
    return pl.pallas_call(
        _scan_body,
        grid=(nblk,),
        in_specs=[
            pl.BlockSpec((8, LOW), lambda i: (0, 0)),
            pl.BlockSpec((bv, LOW), lambda i: (i, 0)),
        ],
        out_specs=pl.BlockSpec((bv, 8), lambda i: (i, 0)),
        out_shape=jax.ShapeDtypeStruct((V, 8), jnp.float32),
    )(means, e_low)


def _sc_gather(table, idx, n_rows):
    info = plsc.get_sparse_core_info()
    nc, ns = info.num_cores, info.num_subcores
    nw = nc * ns
    per = n_rows // nw
    assert per * nw == n_rows and per % 8 == 0
    bat = 32
    nb = per // bat
    assert nb * bat == per
    mesh = plsc.VectorSubcoreMesh(core_axis_name="c", subcore_axis_name="s")

    @functools.partial(
        pl.kernel,
        mesh=mesh,
        out_type=jax.ShapeDtypeStruct((n_rows, D), jnp.float32),
        scratch_types=[
            pltpu.VMEM((per,), jnp.int32),
            pltpu.VMEM((bat, D), jnp.float32),
            pltpu.SemaphoreType.DMA,
        ],
    )
    def k(table_hbm, idx_hbm, out_hbm, idx_v, rows_v, sem):
        wid = lax.axis_index("s") * nc + lax.axis_index("c")
        base = wid * per
        pltpu.sync_copy(idx_hbm.at[pl.ds(base, per)], idx_v)

        def body(j, carry):
            pltpu.async_copy(
                table_hbm.at[idx_v.at[pl.ds(j * bat, bat)]], rows_v, sem
            ).wait()
            pltpu.sync_copy(rows_v, out_hbm.at[pl.ds(base + j * bat, bat)])
            return carry

        lax.fori_loop(0, nb, body, 0)

    return k(table, idx)


def _normalize(x, axis=-1):
    n = jnp.sqrt(jnp.sum(x * x, axis=axis, keepdims=True))
    return x / jnp.maximum(n, 1e-12)


def _nrm(x):
    n = jnp.sqrt(jnp.sum(x * x, axis=1, keepdims=True))
    return x / jnp.maximum(n, 1e-12)


def _loss_body(hs_ref, cand_ref, tgt_ref, idx_ref, tid_ref, ls_ref, out_ref):
    hsb = hs_ref[0]
    hf = _nrm(hsb)
    hl = _nrm(hsb[:, :LOW])

    cand = cand_ref[0]
    candn = _nrm(cand)
    cln = _nrm(cand[:, :LOW])

    hp = lax.Precision.HIGHEST
    dn = (((1,), (1,)), ((), ()))
    full_sims = lax.dot_general(hf, candn, dn, precision=hp,
                                preferred_element_type=jnp.float32)
    low_sims = lax.dot_general(hl, cln, dn, precision=hp,
                               preferred_element_type=jnp.float32)

    tgt = tgt_ref[0]
    tf = jnp.sum(hf * _nrm(tgt), axis=1, keepdims=True)
    tl = jnp.sum(hl * _nrm(tgt[:, :LOW]), axis=1, keepdims=True)

    idx_v = idx_ref[0]
    tid_v = tid_ref[0].reshape(CH, 1)
    is_t = idx_v == tid_v
    neg = jnp.float32(-jnp.inf)
    full_sims = jnp.where(is_t, neg, full_sims)
    low_sims = jnp.where(is_t, neg, low_sims)

    ls = ls_ref[0, 0]
    sc = 100.0 / (1.0 + jnp.exp(-ls)) + 1.0

    def _loss(sims, tsim):
        m = jnp.maximum(jnp.max(sims, axis=1, keepdims=True), tsim)
        z = jnp.sum(jnp.exp((sims - m) * sc), axis=1, keepdims=True)
        z = z + jnp.exp((tsim - m) * sc)
        return jnp.sum(sc * (m - tsim) + jnp.log(z))

    @pl.when(pl.program_id(0) == 0)
    def _():
        out_ref[...] = jnp.zeros_like(out_ref)

    out_ref[...] += _loss(full_sims, tf) + AUX * _loss(low_sims, tl)


def _loss_pallas(hs_chunks, cand, tgt_chunks, idx3, tid3, ls):
    nchunk = hs_chunks.shape[0]
    return pl.pallas_call(
        _loss_body,
        grid=(nchunk,),
        in_specs=[
            pl.BlockSpec((1, CH, D), lambda i: (i, 0, 0)),
            pl.BlockSpec((1, NCAND, D), lambda i: (i, 0, 0)),
            pl.BlockSpec((1, CH, D), lambda i: (i, 0, 0)),
            pl.BlockSpec((1, 1, NCAND), lambda i: (i, 0, 0)),
            pl.BlockSpec((1, 1, CH), lambda i: (i, 0, 0)),
            pl.BlockSpec((1, 1), lambda i: (0, 0)),
        ],
        out_specs=pl.BlockSpec((1, 1), lambda i: (0, 0)),
        out_shape=jax.ShapeDtypeStruct((1, 1), jnp.float32),
    )(hs_chunks, cand, tgt_chunks, idx3, tid3, ls)


def kernel(hidden_states, target_ids, embedding_weight, logit_scale):
    hs = hidden_states.reshape(-1, hidden_states.shape[-1])
    tid = target_ids.reshape(-1)
    total = hs.shape[0]
    nchunk = total // CH

    h_low = _normalize(hs[:, :LOW])
    means = jnp.mean(h_low.reshape(nchunk, CH, LOW), axis=1)

    e_low = embedding_weight[:, :LOW]
    logits = _scan_logits(means, e_low)
    _, top_idx = lax.top_k(logits.T, NCAND)

    all_idx = jnp.concatenate([top_idx.reshape(-1), tid.astype(jnp.int32)])
    rows = _sc_gather(embedding_weight, all_idx, nchunk * NCAND + total)
    cand = rows[: nchunk * NCAND].reshape(nchunk, NCAND, D)
    tgt = rows[nchunk * NCAND:].reshape(nchunk, CH, D)

    tot = _loss_pallas(
        hs.reshape(nchunk, CH, D),
        cand,
        tgt,
        top_idx.reshape(nchunk, 1, NCAND),
        tid.astype(jnp.int32).reshape(nchunk, 1, CH),
        logit_scale.reshape(1, 1),
    )
    return tot[0, 0] / total
